# baseline (speedup 1.0000x reference)
import functools

import jax
import jax.numpy as jnp
from jax import lax
from jax.experimental import pallas as pl
from jax.experimental.pallas import tpu as pltpu
from jax.experimental.pallas import tpu_sc as plsc

VOCAB = 100000
D = 64
N_OUT = 128
B = 4096
L = 50

NC, NS = 2, 16
NW = NC * NS
BPW = B // NW
IPS = 128
NSTEPS = BPW * L // IPS
RING = 8
RROWS = RING * IPS

_mesh = plsc.VectorSubcoreMesh(core_axis_name="c", subcore_axis_name="s")


@functools.partial(
    pl.kernel,
    mesh=_mesh,
    compiler_params=pltpu.CompilerParams(
        use_tc_tiling_on_sc=False, needs_layout_passes=False
    ),
    out_type=jax.ShapeDtypeStruct((B, D), jnp.float32),
    scratch_types=[
        pltpu.VMEM((BPW, L), jnp.int32),
        pltpu.VMEM((NSTEPS, IPS), jnp.int32),
        pltpu.VMEM((RROWS, D), jnp.float32),
        pltpu.VMEM((BPW, D), jnp.float32),
        [pltpu.SemaphoreType.DMA] * RING,
    ],
)
def _sc_pool(idx_hbm, table_hbm, out_hbm, idx_raw, idx_v, rows_v, pool_v, sems):
    wid = lax.axis_index("s") * NC + lax.axis_index("c")

    pltpu.sync_copy(idx_hbm.at[pl.ds(wid * BPW, BPW), 0], idx_raw)

    iota = lax.iota(jnp.int32, 16)

    def repack_row(r, _):
        base = r * L
        for off in (0, 16, 32):
            v = idx_raw[r, pl.ds(off, 16)]
            p = base + off + iota
            plsc.store_scatter(idx_v, [p >> 7, p & 127], v)
        return 0

    lax.fori_loop(0, BPW, repack_row, 0, unroll=4)

    def repack_tail(k, _):
        e = k * 16 + iota
        rows = e >> 1
        cols = 48 + (e & 1)
        v = plsc.load_gather(idx_raw, [rows, cols])
        p = rows * L + cols
        plsc.store_scatter(idx_v, [p >> 7, p & 127], v)
        return 0

    lax.fori_loop(0, BPW * 2 // 16, repack_tail, 0, unroll=4)

    def gather(r, slot):
        return pltpu.make_async_copy(
            table_hbm.at[idx_v.at[r]],
            rows_v.at[pl.ds(slot * IPS, IPS)],
            sems[slot],
        )

    for slot in range(RING - 1):
        gather(slot, slot).start()

    def pool_bag(c):
        base = L * c

        def ld(l, off):
            rr = jnp.bitwise_and(base + l, RROWS - 1)
            return rows_v[rr, pl.ds(off, 16)]

        def body(l, acc):
            return (
                jnp.maximum(acc[0], ld(l, 0)),
                jnp.maximum(acc[1], ld(l, 16)),
                jnp.maximum(acc[2], ld(l, 32)),
                jnp.maximum(acc[3], ld(l, 48)),
            )

        a0, a1, a2, a3 = lax.fori_loop(
            1, L, body, (ld(0, 0), ld(0, 16), ld(0, 32), ld(0, 48)),
            unroll=7,
        )
        pool_v[c, pl.ds(0, 16)] = a0
        pool_v[c, pl.ds(16, 16)] = a1
        pool_v[c, pl.ds(32, 16)] = a2
        pool_v[c, pl.ds(48, 16)] = a3

    def step_work(r, slot, c):
        gather(r, slot).wait()

        pool_bag(c)
        pool_bag(c + 1)
        third = L * (c + 2) + L <= IPS * (r + 1)

        @pl.when(third)
        def _():
            pool_bag(c + 2)

        c = c + jnp.where(third, jnp.int32(3), jnp.int32(2))

        @pl.when(r + RING - 1 < NSTEPS)
        def _():
            gather(r + RING - 1, (slot + RING - 1) % RING).start()

        return c

    def outer(k, c):
        for b in range(RING):
            c = step_work(RING * k + b, b, c)
        return c

    c = lax.fori_loop(0, NSTEPS // RING, outer, jnp.int32(0))
    for r in range(RING * (NSTEPS // RING), NSTEPS):
        c = step_work(jnp.int32(r), r % RING, c)

    pltpu.sync_copy(pool_v, out_hbm.at[pl.ds(wid * BPW, BPW)])


def _mm_body(p_ref, w_ref, o_ref):
    o_ref[:] = lax.dot_general(
        p_ref[:], w_ref[:],
        (((1,), (1,)), ((), ())),
        preferred_element_type=jnp.float32,
    )


def kernel(x, table, W_out):
    pooled = _sc_pool(x.astype(jnp.int32), table)
    out = pl.pallas_call(
        _mm_body,
        out_shape=jax.ShapeDtypeStruct((B, N_OUT), jnp.float32),
    )(pooled, W_out)
    return out


# speedup vs baseline: 1.0650x; 1.0650x over previous
import functools

import jax
import jax.numpy as jnp
from jax import lax
from jax.experimental import pallas as pl
from jax.experimental.pallas import tpu as pltpu
from jax.experimental.pallas import tpu_sc as plsc

VOCAB = 100000
D = 64
N_OUT = 128
B = 4096
L = 50

NC, NS = 2, 16
NW = NC * NS
BPW = B // NW
G = 2
STEPS = BPW // G
NBUF = 4

_mesh = plsc.VectorSubcoreMesh(core_axis_name="c", subcore_axis_name="s")


@functools.partial(
    pl.kernel,
    mesh=_mesh,
    compiler_params=pltpu.CompilerParams(use_tc_tiling_on_sc=False),
    out_type=jax.ShapeDtypeStruct((B, D), jnp.float32),
    scratch_types=[
        pltpu.VMEM((STEPS, G * L), jnp.int32),
        pltpu.VMEM((NBUF, G * L, D), jnp.float32),
        pltpu.VMEM((BPW, D), jnp.float32),
        [pltpu.SemaphoreType.DMA] * NBUF,
    ],
)
def _sc_pool(idx_hbm, table_hbm, out_hbm, idx_v, rows_v, pool_v, sems):
    wid = lax.axis_index("s") * NC + lax.axis_index("c")

    pltpu.sync_copy(idx_hbm.at[pl.ds(wid * STEPS, STEPS)], idx_v)

    def gather(s, b):
        return pltpu.make_async_copy(
            table_hbm.at[idx_v.at[s]], rows_v.at[b], sems[b]
        )

    for b in range(NBUF - 1):
        gather(b, b).start()

    def pool_one_bag(rows, out_row):
        a0 = rows[0, pl.ds(0, 16)]
        a1 = rows[0, pl.ds(16, 16)]
        a2 = rows[0, pl.ds(32, 16)]
        a3 = rows[0, pl.ds(48, 16)]

        def body(l, acc):
            return (
                jnp.maximum(acc[0], rows[l, pl.ds(0, 16)]),
                jnp.maximum(acc[1], rows[l, pl.ds(16, 16)]),
                jnp.maximum(acc[2], rows[l, pl.ds(32, 16)]),
                jnp.maximum(acc[3], rows[l, pl.ds(48, 16)]),
            )

        a0, a1, a2, a3 = lax.fori_loop(
            1, L, body, (a0, a1, a2, a3), unroll=7
        )
        pool_v[out_row, pl.ds(0, 16)] = a0
        pool_v[out_row, pl.ds(16, 16)] = a1
        pool_v[out_row, pl.ds(32, 16)] = a2
        pool_v[out_row, pl.ds(48, 16)] = a3

    def ring_body(k, _):
        for b in range(NBUF):
            s = NBUF * k + b

            @pl.when(s + NBUF - 1 < STEPS)
            def _():
                gather(s + NBUF - 1, (b + NBUF - 1) % NBUF).start()

            gather(s, b).wait()
            for g in range(G):
                pool_one_bag(rows_v.at[b, pl.ds(g * L, L)], s * G + g)

        return 0

    lax.fori_loop(0, STEPS // NBUF, ring_body, 0)

    pltpu.sync_copy(pool_v, out_hbm.at[pl.ds(wid * BPW, BPW)])


def _mm_body(p_ref, w_ref, o_ref):
    o_ref[:] = lax.dot_general(
        p_ref[:], w_ref[:],
        (((1,), (1,)), ((), ())),
        preferred_element_type=jnp.float32,
    )


def kernel(x, table, W_out):
    idx = jnp.reshape(x.astype(jnp.int32), (NW * STEPS, G * L))
    pooled = _sc_pool(idx, table)
    out = pl.pallas_call(
        _mm_body,
        out_shape=jax.ShapeDtypeStruct((B, N_OUT), jnp.float32),
    )(pooled, W_out)
    return out
